# baseline (speedup 1.0000x reference)
import functools

import jax
import jax.numpy as jnp
from jax import lax
from jax.experimental import pallas as pl
from jax.experimental.pallas import tpu as pltpu
from jax.experimental.pallas import tpu_sc as plsc

_B, _C, _N = 8, 3, 2048
_R = 256
_NB = _N // _R
_INF = float("inf")

_NW = 32
_NH = _N // 2
_ROWS_PER_W = (_B * _NH) // _NW
_BLK = 8
_NBLK = _ROWS_PER_W // _BLK


def _pd_body(x_ref, q_ref, pd_ref):
    xb = x_ref[0]
    q = q_ref[0]
    inner = -2.0 * jnp.dot(q, xb, preferred_element_type=jnp.float32)
    xxj = xb[0:1, :] * xb[0:1, :] + xb[1:2, :] * xb[1:2, :] + xb[2:3, :] * xb[2:3, :]
    xxi = q[:, 0:1] * q[:, 0:1] + q[:, 1:2] * q[:, 1:2] + q[:, 2:3] * q[:, 2:3]
    pd_ref[...] = 0.0 - (((0.0 - xxj) - inner) - xxi)


def _pd_call(x0, ptab, half):
    nbh = _NB // 2
    return pl.pallas_call(
        _pd_body,
        grid=(_B, nbh),
        in_specs=[
            pl.BlockSpec((1, _C, _N), lambda b, r: (b, 0, 0)),
            pl.BlockSpec((1, _R, 3), lambda b, r: (b, r + nbh * half, 0)),
        ],
        out_specs=pl.BlockSpec((_R, _N), lambda b, r: (b * nbh + r, 0)),
        out_shape=jax.ShapeDtypeStruct((_B * _N // 2, _N), jnp.float32),
    )(x0, ptab)


def _merge16_asc(a, b):
    rb = lax.rev(b, (0,))
    lo = jnp.minimum(a, rb)
    hi = jnp.maximum(a, rb)
    return jnp.sort(lo), jnp.sort(hi)


def _thresh32(cmins):
    pm = [jnp.minimum(cmins[2 * i], cmins[2 * i + 1]) for i in range(4)]
    s = [jnp.sort(c) for c in pm]
    m01 = _merge16_asc(s[0], s[1])
    m23 = _merge16_asc(s[2], s[3])
    l0 = jnp.minimum(m01[0], lax.rev(m23[1], (0,)))
    l1 = jnp.minimum(m01[1], lax.rev(m23[0], (0,)))
    return jnp.max(jnp.maximum(l0, l1))


def _merge_step(cur, ibuf_h, rbuf, tr):
    c0, c1, i0, i1 = cur
    vi = ibuf_h[pl.ds(tr * 16, 16)]
    v = plsc.load_gather(rbuf, [vi])
    v, vi = plsc.sort_key_val(v, vi)
    rv = lax.rev(v, (0,))
    rvi = lax.rev(vi, (0,))
    m = c1 <= rv
    l1 = jnp.where(m, c1, rv)
    li1 = jnp.where(m, i1, rvi)
    m2 = c0 <= l1
    p = jnp.where(m2, c0, l1)
    pi = jnp.where(m2, i0, li1)
    q = jnp.where(m2, l1, c0)
    qi = jnp.where(m2, li1, i0)
    p, pi = plsc.sort_key_val(p, pi)
    q, qi = plsc.sort_key_val(q, qi)
    return p, q, pi, qi


def _sc_body(pd_hbm, x_hbm, out_hbm, fbuf, rbuf, rbufb, ibuf0, ibuf1, obuf, sema, semb):
    wid = lax.axis_index("s") * 2 + lax.axis_index("c")
    b = wid // 4
    grow = b * _NH + (wid % 4) * _ROWS_PER_W

    pltpu.sync_copy(x_hbm.at[b], fbuf)
    iota = lax.iota(jnp.int32, 16)
    inf16 = jnp.full((16,), _INF, jnp.float32)
    zero16 = jnp.zeros((16,), jnp.int32)
    sent16 = zero16 + (_BLK * _N)
    ibufs = [ibuf0, ibuf1]
    rbuf[pl.ds(_BLK * _N, 16)] = inf16
    rbufb[pl.ds(_BLK * _N, 16)] = inf16
    lag = 4
    nchunk = _N // 16

    def _issue(blk, rb, sem):
        pltpu.async_copy(
            pd_hbm.at[pl.ds((grow + blk * _BLK) * _N, _BLK * _N)],
            rb.at[pl.ds(0, _BLK * _N)], sem)

    _issue(0, rbuf, sema)
    _issue(1, rbufb, semb)

    def blk_body(g2, carry):
        for half, (rb, sem) in enumerate(((rbuf, sema), (rbufb, semb))):
            blk = g2 * 2 + half
            pltpu.make_async_copy(
                pd_hbm.at[pl.ds(0, _BLK * _N)],
                rb.at[pl.ds(0, _BLK * _N)], sem).wait()
            carry = _process_block(blk, rb, carry)

            @pl.when(blk + 2 < _NBLK)
            def _():
                _issue(blk + 2, rb, sem)
        return carry

    def _process_block(blk, rbuf, carry):
        def pair_body(rp, carry2):
            rows = [rp * 2, rp * 2 + 1]
            rbases = [r * _N for r in rows]

            ts = []
            for h in range(2):
                cmins = []
                for g in range(8):
                    acc = rbuf[pl.ds(rbases[h] + g * 256, 16)]
                    for k in range(1, 16):
                        acc = jnp.minimum(
                            acc, rbuf[pl.ds(rbases[h] + g * 256 + k * 16, 16)])
                    cmins.append(acc)
                ts.append(_thresh32(cmins))

            for h in range(2):
                for c5 in range(5):
                    ibufs[h][pl.ds(c5 * 16, 16)] = sent16

            init = []
            for h in range(2):
                vs, pcs = [], []
                for c in range(lag):
                    v = rbuf[pl.ds(rbases[h] + c * 16, 16)]
                    vs.append(v)
                    pcs.append(plsc.all_reduce_population_count(v <= ts[h])[0])
                init.append((jnp.int32(0), tuple(vs), tuple(pcs)))

            @plsc.parallel_loop(lag, nchunk, 1, unroll=8, carry=tuple(init))
            def _pass2(c, carry):
                out = []
                for h in range(2):
                    off, vs, pcs = carry[h]
                    plsc.store_compressed(
                        ibufs[h].at[pl.ds(off, 16)],
                        rbases[h] + (c - lag) * 16 + iota, mask=vs[0] <= ts[h])
                    v = rbuf[pl.ds(rbases[h] + c * 16, 16)]
                    pc = plsc.all_reduce_population_count(v <= ts[h])[0]
                    out.append((off + pcs[0], vs[1:] + (v,), pcs[1:] + (pc,)))
                return tuple(out)

            cnts = []
            for h in range(2):
                off, vs, pcs = _pass2[h]
                for k in range(lag):
                    cc = nchunk - lag + k
                    plsc.store_compressed(
                        ibufs[h].at[pl.ds(off, 16)],
                        rbases[h] + cc * 16 + iota, mask=vs[k] <= ts[h])
                    off = off + pcs[k]
                ibufs[h][pl.ds(off, 16)] = sent16
                cnts.append(off)

            st = [(inf16, inf16, zero16, zero16) for _ in range(2)]
            for tr in range(4):
                for h in range(2):
                    st[h] = _merge_step(st[h], ibufs[h], rbuf, tr)
            for h in range(2):
                ntrip = (cnts[h] + 15) // 16

                def mstep(tr, cur, _h=h):
                    return _merge_step(cur, ibufs[_h], rbuf, tr)

                st[h] = lax.fori_loop(4, ntrip, mstep, st[h])

            lo8 = iota < 8
            zf = jnp.float32(0.0)
            for h in range(2):
                i0 = st[h][2] - rbases[h]
                i1 = st[h][3] - rbases[h]
                g0x = plsc.load_gather(fbuf, [i0])
                g0y = plsc.load_gather(fbuf, [i0 + 2048])
                g0z = plsc.load_gather(fbuf, [i0 + 4096])
                g1x = plsc.load_gather(fbuf, [i1])
                g1y = plsc.load_gather(fbuf, [i1 + 2048])
                g1z = plsc.load_gather(fbuf, [i1 + 4096])
                si = i0 * 3
                s16x = plsc.load_gather(fbuf, [si])
                s16y = plsc.load_gather(fbuf, [si + 1])
                s16z = plsc.load_gather(fbuf, [si + 2])
                s0x, s0y, s0z = jnp.sum(g0x), jnp.sum(g0y), jnp.sum(g0z)
                vals = (
                    jnp.sum(jnp.where(lo8, g0x, zf)) * 0.125,
                    jnp.sum(jnp.where(lo8, g0y, zf)) * 0.125,
                    jnp.sum(jnp.where(lo8, g0z, zf)) * 0.125,
                    jnp.sum(s16x) * 0.0625,
                    jnp.sum(s16y) * 0.0625,
                    jnp.sum(s16z) * 0.0625,
                    (s0x + jnp.sum(g1x)) * 0.03125,
                    (s0y + jnp.sum(g1y)) * 0.03125,
                    (s0z + jnp.sum(g1z)) * 0.03125,
                )
                ovec = jnp.zeros((16,), jnp.float32)
                for c, v in enumerate(vals):
                    ovec = jnp.where(iota == c, v, ovec)
                obuf[pl.ds((blk * _BLK + rows[h]) * 16, 16)] = ovec
            return carry2

        return lax.fori_loop(0, _BLK // 2, pair_body, carry)

    lax.fori_loop(0, _NBLK // 2, blk_body, jnp.int32(0))
    pltpu.sync_copy(obuf, out_hbm.at[pl.ds(grow * 16, _ROWS_PER_W * 16)])


def _make_sc_kernel():
    @functools.partial(
        pl.kernel,
        out_type=jax.ShapeDtypeStruct((_B * _NH * 16,), jnp.float32),
        mesh=plsc.VectorSubcoreMesh(core_axis_name="c", subcore_axis_name="s"),
        compiler_params=pltpu.CompilerParams(needs_layout_passes=False),
        scratch_types=[
            pltpu.VMEM((_C * _N,), jnp.float32),
            pltpu.VMEM((_BLK * _N + 16,), jnp.float32),
            pltpu.VMEM((_BLK * _N + 16,), jnp.float32),
            pltpu.VMEM((_N + 32,), jnp.int32),
            pltpu.VMEM((_N + 32,), jnp.int32),
            pltpu.VMEM((_ROWS_PER_W * 16,), jnp.float32),
            pltpu.SemaphoreType.DMA,
            pltpu.SemaphoreType.DMA,
        ],
    )
    def _sc_knn(pd_hbm, x_hbm, out_hbm, fbuf, rbuf, rbufb, ibuf0, ibuf1,
                obuf, sema, semb):
        _sc_body(pd_hbm, x_hbm, out_hbm, fbuf, rbuf, rbufb, ibuf0, ibuf1,
                 obuf, sema, semb)

    return _sc_knn


_sc_knn = _make_sc_kernel()


def kernel(x):
    batch_size = x.shape[0]
    num_points = x.shape[3]
    x0 = jnp.reshape(x, (batch_size, -1, num_points))
    ptab = jnp.swapaxes(x0, 1, 2)

    xflat = jnp.reshape(x0, (batch_size, _C * _N))
    halves = []
    for h in range(2):
        sdist = _pd_call(x0, ptab, h)
        fh = _sc_knn(jnp.reshape(sdist, (_B * _NH * _N,)), xflat)
        halves.append(fh.reshape(batch_size, _NH, 16))
    feats = jnp.concatenate(halves, axis=1)
    f8, f16, f32 = feats[..., 0:3], feats[..., 3:6], feats[..., 6:9]

    concat_x = jnp.swapaxes(jnp.expand_dims(x0, 3), 2, 1)
    for feat in (f8, f16, f32):
        feature = feat.reshape(batch_size, num_points, 1, 1, 3)
        num_dims = concat_x.shape[3]
        concat_x = jnp.reshape(concat_x, (batch_size, num_points, 1, num_dims, 3))
        concat_x = jnp.concatenate((feature, concat_x), axis=3)
        concat_x = jnp.transpose(concat_x, (0, 4, 1, 3, 2))
    concat_x = jnp.transpose(concat_x, (0, 3, 1, 2, 4))
    return concat_x


# speedup vs baseline: 1.0144x; 1.0144x over previous
import functools

import jax
import jax.numpy as jnp
from jax import lax
from jax.experimental import pallas as pl
from jax.experimental.pallas import tpu as pltpu
from jax.experimental.pallas import tpu_sc as plsc

_B, _C, _N = 8, 3, 2048
_R = 256
_NB = _N // _R
_INF = float("inf")

_NW = 32
_ROWS_PER_W = (_B * _N) // _NW
_BLK = 16
_NBLK = _ROWS_PER_W // _BLK


def _pd_body(x_ref, q_ref, pd_ref):
    xb = x_ref[0]
    q = q_ref[0]
    inner = -2.0 * jnp.dot(q, xb, preferred_element_type=jnp.float32)
    xxj = xb[0:1, :] * xb[0:1, :] + xb[1:2, :] * xb[1:2, :] + xb[2:3, :] * xb[2:3, :]
    xxi = q[:, 0:1] * q[:, 0:1] + q[:, 1:2] * q[:, 1:2] + q[:, 2:3] * q[:, 2:3]
    pd_ref[...] = 0.0 - (((0.0 - xxj) - inner) - xxi)


def _pd_call(x0, ptab):
    return pl.pallas_call(
        _pd_body,
        grid=(_B, _NB),
        in_specs=[
            pl.BlockSpec((1, _C, _N), lambda b, r: (b, 0, 0)),
            pl.BlockSpec((1, _R, 3), lambda b, r: (b, r, 0)),
        ],
        out_specs=pl.BlockSpec((_R, _N), lambda b, r: (b * _NB + r, 0)),
        out_shape=jax.ShapeDtypeStruct((_B * _N, _N), jnp.float32),
    )(x0, ptab)


def _merge16_asc(a, b):
    rb = lax.rev(b, (0,))
    lo = jnp.minimum(a, rb)
    hi = jnp.maximum(a, rb)
    return jnp.sort(lo), jnp.sort(hi)


def _thresh32(cmins):
    pm = [jnp.minimum(cmins[2 * i], cmins[2 * i + 1]) for i in range(4)]
    s = [jnp.sort(c) for c in pm]
    m01 = _merge16_asc(s[0], s[1])
    m23 = _merge16_asc(s[2], s[3])
    l0 = jnp.minimum(m01[0], lax.rev(m23[1], (0,)))
    l1 = jnp.minimum(m01[1], lax.rev(m23[0], (0,)))
    return jnp.max(jnp.maximum(l0, l1))


def _merge_step(cur, ibuf_h, rbuf, tr):
    c0, c1, i0, i1 = cur
    vi = ibuf_h[pl.ds(tr * 16, 16)]
    v = plsc.load_gather(rbuf, [vi])
    v, vi = plsc.sort_key_val(v, vi)
    rv = lax.rev(v, (0,))
    rvi = lax.rev(vi, (0,))
    m = c1 <= rv
    l1 = jnp.where(m, c1, rv)
    li1 = jnp.where(m, i1, rvi)
    m2 = c0 <= l1
    p = jnp.where(m2, c0, l1)
    pi = jnp.where(m2, i0, li1)
    q = jnp.where(m2, l1, c0)
    qi = jnp.where(m2, li1, i0)
    p, pi = plsc.sort_key_val(p, pi)
    q, qi = plsc.sort_key_val(q, qi)
    return p, q, pi, qi


def _sc_body(pd_hbm, x_hbm, out_hbm, fbuf, rbuf, rbufb, ibuf0, ibuf1, obuf, sema, semb):
    wid = lax.axis_index("s") * 2 + lax.axis_index("c")
    b = wid // 4
    grow = b * _N + (wid % 4) * _ROWS_PER_W

    pltpu.sync_copy(x_hbm.at[b], fbuf)
    iota = lax.iota(jnp.int32, 16)
    inf16 = jnp.full((16,), _INF, jnp.float32)
    zero16 = jnp.zeros((16,), jnp.int32)
    sent16 = zero16 + (_BLK * _N)
    ibufs = [ibuf0, ibuf1]
    rbuf[pl.ds(_BLK * _N, 16)] = inf16
    rbufb[pl.ds(_BLK * _N, 16)] = inf16
    lag = 4
    nchunk = _N // 16

    def _issue(blk, rb, sem):
        pltpu.async_copy(
            pd_hbm.at[pl.ds((grow + blk * _BLK) * _N, _BLK * _N)],
            rb.at[pl.ds(0, _BLK * _N)], sem)

    _issue(0, rbuf, sema)
    _issue(1, rbufb, semb)

    def blk_body(g2, carry):
        for half, (rb, sem) in enumerate(((rbuf, sema), (rbufb, semb))):
            blk = g2 * 2 + half
            pltpu.make_async_copy(
                pd_hbm.at[pl.ds(0, _BLK * _N)],
                rb.at[pl.ds(0, _BLK * _N)], sem).wait()
            carry = _process_block(blk, rb, carry)

            @pl.when(blk + 2 < _NBLK)
            def _():
                _issue(blk + 2, rb, sem)
        return carry

    def _process_block(blk, rbuf, carry):
        def pair_body(rp, carry2):
            rows = [rp * 2, rp * 2 + 1]
            rbases = [r * _N for r in rows]

            ts = []
            for h in range(2):
                cmins = []
                for g in range(8):
                    acc = rbuf[pl.ds(rbases[h] + g * 256, 16)]
                    for k in range(1, 16):
                        acc = jnp.minimum(
                            acc, rbuf[pl.ds(rbases[h] + g * 256 + k * 16, 16)])
                    cmins.append(acc)
                ts.append(_thresh32(cmins))

            for h in range(2):
                for c5 in range(5):
                    ibufs[h][pl.ds(c5 * 16, 16)] = sent16

            init = []
            for h in range(2):
                vs, pcs = [], []
                for c in range(lag):
                    v = rbuf[pl.ds(rbases[h] + c * 16, 16)]
                    vs.append(v)
                    pcs.append(plsc.all_reduce_population_count(v <= ts[h])[0])
                init.append((jnp.int32(0), tuple(vs), tuple(pcs)))

            @plsc.parallel_loop(lag, nchunk, 1, unroll=8, carry=tuple(init))
            def _pass2(c, carry):
                out = []
                for h in range(2):
                    off, vs, pcs = carry[h]
                    plsc.store_compressed(
                        ibufs[h].at[pl.ds(off, 16)],
                        rbases[h] + (c - lag) * 16 + iota, mask=vs[0] <= ts[h])
                    v = rbuf[pl.ds(rbases[h] + c * 16, 16)]
                    pc = plsc.all_reduce_population_count(v <= ts[h])[0]
                    out.append((off + pcs[0], vs[1:] + (v,), pcs[1:] + (pc,)))
                return tuple(out)

            cnts = []
            for h in range(2):
                off, vs, pcs = _pass2[h]
                for k in range(lag):
                    cc = nchunk - lag + k
                    plsc.store_compressed(
                        ibufs[h].at[pl.ds(off, 16)],
                        rbases[h] + cc * 16 + iota, mask=vs[k] <= ts[h])
                    off = off + pcs[k]
                ibufs[h][pl.ds(off, 16)] = sent16
                cnts.append(off)

            st = [(inf16, inf16, zero16, zero16) for _ in range(2)]
            for tr in range(4):
                for h in range(2):
                    st[h] = _merge_step(st[h], ibufs[h], rbuf, tr)
            for h in range(2):
                ntrip = (cnts[h] + 15) // 16

                def mstep(tr, cur, _h=h):
                    return _merge_step(cur, ibufs[_h], rbuf, tr)

                st[h] = lax.fori_loop(4, ntrip, mstep, st[h])

            lo8 = iota < 8
            zf = jnp.float32(0.0)
            for h in range(2):
                i0 = st[h][2] - rbases[h]
                i1 = st[h][3] - rbases[h]
                g0x = plsc.load_gather(fbuf, [i0])
                g0y = plsc.load_gather(fbuf, [i0 + 2048])
                g0z = plsc.load_gather(fbuf, [i0 + 4096])
                g1x = plsc.load_gather(fbuf, [i1])
                g1y = plsc.load_gather(fbuf, [i1 + 2048])
                g1z = plsc.load_gather(fbuf, [i1 + 4096])
                si = i0 * 3
                s16x = plsc.load_gather(fbuf, [si])
                s16y = plsc.load_gather(fbuf, [si + 1])
                s16z = plsc.load_gather(fbuf, [si + 2])
                s0x, s0y, s0z = jnp.sum(g0x), jnp.sum(g0y), jnp.sum(g0z)
                vals = (
                    jnp.sum(jnp.where(lo8, g0x, zf)) * 0.125,
                    jnp.sum(jnp.where(lo8, g0y, zf)) * 0.125,
                    jnp.sum(jnp.where(lo8, g0z, zf)) * 0.125,
                    jnp.sum(s16x) * 0.0625,
                    jnp.sum(s16y) * 0.0625,
                    jnp.sum(s16z) * 0.0625,
                    (s0x + jnp.sum(g1x)) * 0.03125,
                    (s0y + jnp.sum(g1y)) * 0.03125,
                    (s0z + jnp.sum(g1z)) * 0.03125,
                )
                ovec = jnp.zeros((16,), jnp.float32)
                for c, v in enumerate(vals):
                    ovec = jnp.where(iota == c, v, ovec)
                obuf[pl.ds((blk * _BLK + rows[h]) * 16, 16)] = ovec
            return carry2

        return lax.fori_loop(0, _BLK // 2, pair_body, carry)

    lax.fori_loop(0, _NBLK // 2, blk_body, jnp.int32(0))
    pltpu.sync_copy(obuf, out_hbm.at[pl.ds(grow * 16, _ROWS_PER_W * 16)])


@functools.partial(
    pl.kernel,
    out_type=jax.ShapeDtypeStruct((_B * _N * 16,), jnp.float32),
    mesh=plsc.VectorSubcoreMesh(core_axis_name="c", subcore_axis_name="s"),
    compiler_params=pltpu.CompilerParams(needs_layout_passes=False),
    scratch_types=[
        pltpu.VMEM((_C * _N,), jnp.float32),
        pltpu.VMEM((_BLK * _N + 16,), jnp.float32),
        pltpu.VMEM((_BLK * _N + 16,), jnp.float32),
        pltpu.VMEM((_N + 32,), jnp.int32),
        pltpu.VMEM((_N + 32,), jnp.int32),
        pltpu.VMEM((_ROWS_PER_W * 16,), jnp.float32),
        pltpu.SemaphoreType.DMA,
        pltpu.SemaphoreType.DMA,
    ],
)
def _sc_knn(pd_hbm, x_hbm, out_hbm, fbuf, rbuf, rbufb, ibuf0, ibuf1, obuf, sema, semb):
    _sc_body(pd_hbm, x_hbm, out_hbm, fbuf, rbuf, rbufb, ibuf0, ibuf1, obuf, sema, semb)


def kernel(x):
    batch_size = x.shape[0]
    num_points = x.shape[3]
    x0 = jnp.reshape(x, (batch_size, -1, num_points))
    ptab = jnp.swapaxes(x0, 1, 2)

    sdist = _pd_call(x0, ptab)
    feats = _sc_knn(jnp.reshape(sdist, (_B * _N * _N,)),
                    jnp.reshape(x0, (batch_size, _C * _N)))
    feats = feats.reshape(batch_size, num_points, 16)
    f8, f16, f32 = feats[..., 0:3], feats[..., 3:6], feats[..., 6:9]

    concat_x = jnp.swapaxes(jnp.expand_dims(x0, 3), 2, 1)
    for feat in (f8, f16, f32):
        feature = feat.reshape(batch_size, num_points, 1, 1, 3)
        num_dims = concat_x.shape[3]
        concat_x = jnp.reshape(concat_x, (batch_size, num_points, 1, num_dims, 3))
        concat_x = jnp.concatenate((feature, concat_x), axis=3)
        concat_x = jnp.transpose(concat_x, (0, 4, 1, 3, 2))
    concat_x = jnp.transpose(concat_x, (0, 3, 1, 2, 4))
    return concat_x
